# baseline (speedup 1.0000x reference)
import functools

import jax
import jax.numpy as jnp
from jax import lax
from jax.experimental import pallas as pl
from jax.experimental.pallas import tpu as pltpu
from jax.experimental.pallas import tpu_sc as plsc

B, N, S, D = 4, 32, 64, 128
T = 6 + (N - 6) * 36
NC, NS = 2, 16
NW = NC * NS
NI = N - 6
WPB = NW // B
MAXPEND = 24


def _body(x_hbm, out_hbm, hot, cold, sem):
    cid = lax.axis_index("c")
    sid = lax.axis_index("s")
    wid = sid * NC + cid
    bi = wid // WPB
    r = wid % WPB
    ra = r
    rb = (r + 4) % WPB

    stage = []

    stage.append(pltpu.async_copy(
        x_hbm.at[pl.ds(bi * N + 26, 6)], hot, sem))

    for base, slot0 in ((ra, 0), (rb, 4)):
        for k in range(3):
            stage.append(pltpu.async_copy(
                x_hbm.at[pl.ds(bi * N + base + 8 * k, 1)],
                cold.at[pl.ds(slot0 + k, 1)], sem))

        @pl.when(base < NI - 3 * WPB)
        def _(base=base, slot0=slot0):
            pltpu.sync_copy(
                x_hbm.at[pl.ds(bi * N + base + 24, 1)],
                cold.at[pl.ds(slot0 + 3, 1)])
    for d in stage:
        d.wait()

    pending = []

    def fire(src, dst):
        while len(pending) >= MAXPEND:
            pending.pop(0).wait()
        pending.append(pltpu.async_copy(src, dst, sem))

    @pl.when(r == 2 + bi % 2)
    def _():
        pltpu.sync_copy(hot, out_hbm.at[pl.ds(bi * T, 6)])

    for g in range(6):
        j0 = ra if g % 2 == 0 else rb
        slot0 = 0 if g % 2 == 0 else 4

        def do_item(i, k, copy):
            dst0 = bi * T + 6 + 36 * i + 6 * g
            if g > 0:
                copy(hot.at[pl.ds(0, g)],
                     out_hbm.at[pl.ds(dst0, g)])
            copy(cold.at[pl.ds(slot0 + k, 1)],
                 out_hbm.at[pl.ds(dst0 + g, 1)])
            if g < 5:
                copy(hot.at[pl.ds(g + 1, 5 - g)],
                     out_hbm.at[pl.ds(dst0 + g + 1, 5 - g)])

        for k in range(3):
            do_item(j0 + WPB * k, k, fire)

        @pl.when(j0 < NI - 3 * WPB)
        def _():
            local = []
            do_item(j0 + WPB * 3, 3,
                    lambda s_, d_: local.append(pltpu.async_copy(s_, d_, sem)))
            for d_ in local:
                d_.wait()

    for d in pending:
        d.wait()


@functools.partial(
    pl.kernel,
    out_type=jax.ShapeDtypeStruct((B * T, S, D), jnp.float32),
    mesh=plsc.VectorSubcoreMesh(core_axis_name="c", subcore_axis_name="s"),
    scratch_types=[
        pltpu.VMEM((6, S, D), jnp.float32),
        pltpu.VMEM((8, S, D), jnp.float32),
        pltpu.SemaphoreType.DMA,
    ],
)
def _gather_rows(x_hbm, out_hbm, hot, cold, sem):
    _body(x_hbm, out_hbm, hot, cold, sem)


def kernel(x):
    b, n, s, d = x.shape
    out = _gather_rows(x.reshape(b * n, s, d))
    return out.reshape(b, 6, T // 6, s, d)


# speedup vs baseline: 1.0040x; 1.0040x over previous
import functools

import jax
import jax.numpy as jnp
from jax import lax
from jax.experimental import pallas as pl
from jax.experimental.pallas import tpu as pltpu
from jax.experimental.pallas import tpu_sc as plsc

B, N, S, D = 4, 32, 64, 128
T = 6 + (N - 6) * 36
NC, NS = 2, 16
NW = NC * NS
NI = N - 6
WPB = NW // B
MAXPEND = 16


def _body(x_hbm, out_hbm, hot, cold, sem):
    cid = lax.axis_index("c")
    sid = lax.axis_index("s")
    wid = sid * NC + cid
    bi = wid // WPB
    r = wid % WPB
    ra = r
    rb = (r + 4) % WPB

    stage = []

    stage.append(pltpu.async_copy(
        x_hbm.at[pl.ds(bi * N + 26, 6)], hot, sem))

    for base, slot0 in ((ra, 0), (rb, 4)):
        for k in range(3):
            stage.append(pltpu.async_copy(
                x_hbm.at[pl.ds(bi * N + base + 8 * k, 1)],
                cold.at[pl.ds(slot0 + k, 1)], sem))

        @pl.when(base < NI - 3 * WPB)
        def _(base=base, slot0=slot0):
            pltpu.sync_copy(
                x_hbm.at[pl.ds(bi * N + base + 24, 1)],
                cold.at[pl.ds(slot0 + 3, 1)])
    for d in stage:
        d.wait()

    pending = []

    def fire(src, dst):
        while len(pending) >= MAXPEND:
            pending.pop(0).wait()
        pending.append(pltpu.async_copy(src, dst, sem))

    @pl.when(r == bi % 2)
    def _():
        pltpu.sync_copy(hot, out_hbm.at[pl.ds(bi * T, 6)])

    for g in range(6):
        j0 = ra if g % 2 == 0 else rb
        slot0 = 0 if g % 2 == 0 else 4

        def do_item(i, k, copy):
            dst0 = bi * T + 6 + 36 * i + 6 * g
            if g > 0:
                copy(hot.at[pl.ds(0, g)],
                     out_hbm.at[pl.ds(dst0, g)])
            copy(cold.at[pl.ds(slot0 + k, 1)],
                 out_hbm.at[pl.ds(dst0 + g, 1)])
            if g < 5:
                copy(hot.at[pl.ds(g + 1, 5 - g)],
                     out_hbm.at[pl.ds(dst0 + g + 1, 5 - g)])

        for k in range(3):
            do_item(j0 + WPB * k, k, fire)

        @pl.when(j0 < NI - 3 * WPB)
        def _():
            i = j0 + WPB * 3
            pltpu.sync_copy(cold.at[pl.ds(slot0 + 3, 1)],
                            out_hbm.at[pl.ds(bi * T + 6 + 36 * i + 6 * g + g,
                                             1)])

        for e in range(NI - 3 * WPB):
            rt = (2, 3, 6, 7)[(2 * g + e) % 4]

            @pl.when(r == rt)
            def _(e=e):
                i = e + WPB * 3
                dst0 = bi * T + 6 + 36 * i + 6 * g
                local = []
                if g > 0:
                    local.append(pltpu.async_copy(
                        hot.at[pl.ds(0, g)],
                        out_hbm.at[pl.ds(dst0, g)], sem))
                if g < 5:
                    local.append(pltpu.async_copy(
                        hot.at[pl.ds(g + 1, 5 - g)],
                        out_hbm.at[pl.ds(dst0 + g + 1, 5 - g)], sem))
                for d_ in local:
                    d_.wait()

    for d in pending:
        d.wait()


@functools.partial(
    pl.kernel,
    out_type=jax.ShapeDtypeStruct((B * T, S, D), jnp.float32),
    mesh=plsc.VectorSubcoreMesh(core_axis_name="c", subcore_axis_name="s"),
    scratch_types=[
        pltpu.VMEM((6, S, D), jnp.float32),
        pltpu.VMEM((8, S, D), jnp.float32),
        pltpu.SemaphoreType.DMA,
    ],
)
def _gather_rows(x_hbm, out_hbm, hot, cold, sem):
    _body(x_hbm, out_hbm, hot, cold, sem)


def kernel(x):
    b, n, s, d = x.shape
    out = _gather_rows(x.reshape(b * n, s, d))
    return out.reshape(b, 6, T // 6, s, d)
